# baseline (speedup 1.0000x reference)
import functools

import jax
import jax.numpy as jnp
from jax import lax
from jax.experimental import pallas as pl
from jax.experimental.pallas import tpu as pltpu
from jax.experimental.pallas import tpu_sc as plsc

CHUNK = 400
SUB = 80
NSUB = CHUNK // SUB
DPAD = 96


def kernel(x, atom_fea):
    B = x.shape[0]
    V, D = atom_fea.shape
    tab = jnp.pad(atom_fea, ((0, 0), (0, DPAD - D)))
    n_chunks = B // CHUNK
    assert n_chunks * CHUNK == B

    info = plsc.get_sparse_core_info()
    nw = info.num_cores * info.num_subcores
    ni = (n_chunks + nw - 1) // nw
    mesh = plsc.VectorSubcoreMesh(core_axis_name="c", subcore_axis_name="s")

    @functools.partial(
        pl.kernel,
        mesh=mesh,
        out_type=jax.ShapeDtypeStruct((B, DPAD), jnp.float32),
        scratch_types=[
            pltpu.VMEM((2, CHUNK), jnp.int32),
            pltpu.VMEM((2, CHUNK, DPAD), jnp.float32),
            pltpu.SemaphoreType.DMA,
            pltpu.SemaphoreType.DMA,
            pltpu.SemaphoreType.DMA,
            pltpu.SemaphoreType.DMA,
        ],
        compiler_params=pltpu.CompilerParams(use_tc_tiling_on_sc=False),
    )
    def gather_kernel(x_hbm, tab_hbm, out_hbm, idx_v, rows_v, sem_i, sg0, sg1, sem_o):
        c = lax.axis_index("c")
        s = lax.axis_index("s")
        wid = s * info.num_cores + c
        sem_g = [sg0, sg1]

        def base_of(k):
            return (wid + k * nw) * CHUNK

        def valid(k):
            return wid + k * nw < n_chunks

        def idx_copy(k):
            return pltpu.make_async_copy(
                x_hbm.at[pl.ds(base_of(k), CHUNK)], idx_v.at[k % 2], sem_i
            )

        def gather_copies(k):
            b = k % 2
            return [
                pltpu.make_async_copy(
                    tab_hbm.at[idx_v.at[b].at[pl.ds(j * SUB, SUB)]],
                    rows_v.at[b].at[pl.ds(j * SUB, SUB)],
                    sem_g[b],
                )
                for j in range(NSUB)
            ]

        def out_copy(k):
            return pltpu.make_async_copy(
                rows_v.at[k % 2], out_hbm.at[pl.ds(base_of(k), CHUNK)], sem_o
            )

        def when_start(k, cp):
            @pl.when(valid(k))
            def _():
                cp.start()

        def when_wait(k, cps):
            @pl.when(valid(k))
            def _():
                for cp in cps:
                    cp.wait()

        when_start(0, idx_copy(0))
        when_wait(0, [idx_copy(0)])
        if ni > 1:
            when_start(1, idx_copy(1))
        for cp in gather_copies(0):
            when_start(0, cp)

        for i in range(ni):
            when_wait(i, gather_copies(i))
            when_start(i, out_copy(i))
            if i >= 1:
                when_wait(i - 1, [out_copy(i - 1)])
            if i + 1 < ni:
                when_wait(i + 1, [idx_copy(i + 1)])
                for cp in gather_copies(i + 1):
                    when_start(i + 1, cp)
            if i + 2 < ni:
                when_start(i + 2, idx_copy(i + 2))

        when_wait(ni - 1, [out_copy(ni - 1)])

    return gather_kernel(x, tab)[:, :D]


# speedup vs baseline: 1.0150x; 1.0150x over previous
import functools

import jax
import jax.numpy as jnp
from jax import lax
from jax.experimental import pallas as pl
from jax.experimental.pallas import tpu as pltpu
from jax.experimental.pallas import tpu_sc as plsc

CHUNK = 400
DPAD = 96
NVR = DPAD // 16
UNROLL = 4


def kernel(x, atom_fea):
    B = x.shape[0]
    V, D = atom_fea.shape
    tab = jnp.pad(atom_fea, ((0, 0), (0, DPAD - D)))
    n_chunks = B // CHUNK
    assert n_chunks * CHUNK == B

    info = plsc.get_sparse_core_info()
    nw = info.num_cores * info.num_subcores
    ni = (n_chunks + nw - 1) // nw
    mesh = plsc.VectorSubcoreMesh(core_axis_name="c", subcore_axis_name="s")

    @functools.partial(
        pl.kernel,
        mesh=mesh,
        out_type=jax.ShapeDtypeStruct((B, DPAD), jnp.float32),
        scratch_types=[
            pltpu.VMEM((V, DPAD), jnp.float32),
            pltpu.VMEM((2, CHUNK), jnp.int32),
            pltpu.VMEM((2, CHUNK, DPAD), jnp.float32),
            pltpu.SemaphoreType.DMA,
            pltpu.SemaphoreType.DMA,
            pltpu.SemaphoreType.DMA,
        ],
        compiler_params=pltpu.CompilerParams(
            use_tc_tiling_on_sc=False, needs_layout_passes=False
        ),
    )
    def gather_kernel(x_hbm, tab_hbm, out_hbm, tab_v, idx_v, rows_v, sem_t, sem_i, sem_o):
        c = lax.axis_index("c")
        s = lax.axis_index("s")
        wid = s * info.num_cores + c
        lanes = lax.iota(jnp.int32, 16)
        cols = [16 * j + lanes for j in range(NVR)]

        def base_of(k):
            return (wid + k * nw) * CHUNK

        def valid(k):
            return wid + k * nw < n_chunks

        def idx_copy(k):
            return pltpu.make_async_copy(
                x_hbm.at[pl.ds(base_of(k), CHUNK)], idx_v.at[k % 2], sem_i
            )

        def out_copy(k):
            return pltpu.make_async_copy(
                rows_v.at[k % 2], out_hbm.at[pl.ds(base_of(k), CHUNK)], sem_o
            )

        def when_start(k, cp):
            @pl.when(valid(k))
            def _():
                cp.start()

        def when_wait(k, cps):
            @pl.when(valid(k))
            def _():
                for cp in cps:
                    cp.wait()

        tab_cp = pltpu.make_async_copy(tab_hbm, tab_v, sem_t)
        tab_cp.start()
        when_start(0, idx_copy(0))
        tab_cp.wait()
        when_wait(0, [idx_copy(0)])
        if ni > 1:
            when_start(1, idx_copy(1))

        for i in range(ni):
            b = i % 2

            @pl.when(valid(i))
            def _(b=b):
                idx_b = idx_v.at[b]
                rows_b = rows_v.at[b]

                def body(g, carry):
                    a0 = g * UNROLL
                    for u in range(UNROLL):
                        a = a0 + u
                        xa = plsc.load_gather(idx_b, [lanes * 0 + a])
                        for j in range(NVR):
                            v = plsc.load_gather(tab_v, [xa, cols[j]])
                            rows_b.at[a][pl.ds(16 * j, 16)] = v
                    return carry

                lax.fori_loop(0, CHUNK // UNROLL, body, 0)

            if i >= 1:
                when_wait(i - 1, [out_copy(i - 1)])
            when_start(i, out_copy(i))
            if i + 1 < ni:
                when_wait(i + 1, [idx_copy(i + 1)])
            if i + 2 < ni:
                when_start(i + 2, idx_copy(i + 2))

        when_wait(ni - 1, [out_copy(ni - 1)])

    return gather_kernel(x, tab)[:, :D]


# speedup vs baseline: 1.1238x; 1.1073x over previous
import functools

import jax
import jax.numpy as jnp
from jax import lax
from jax.experimental import pallas as pl
from jax.experimental.pallas import tpu as pltpu
from jax.experimental.pallas import tpu_sc as plsc

CHUNK = 400
DPAD = 96
NVR = DPAD // 16
UNROLL = 8


def kernel(x, atom_fea):
    B = x.shape[0]
    V, D = atom_fea.shape
    tab = jnp.pad(atom_fea, ((0, 0), (0, DPAD - D)))
    n_chunks = B // CHUNK
    assert n_chunks * CHUNK == B

    info = plsc.get_sparse_core_info()
    nw = info.num_cores * info.num_subcores
    ni = (n_chunks + nw - 1) // nw
    mesh = plsc.VectorSubcoreMesh(core_axis_name="c", subcore_axis_name="s")

    @functools.partial(
        pl.kernel,
        mesh=mesh,
        out_type=jax.ShapeDtypeStruct((B, DPAD), jnp.float32),
        scratch_types=[
            pltpu.VMEM((V, DPAD), jnp.float32),
            pltpu.VMEM((2, CHUNK), jnp.int32),
            pltpu.VMEM((2, CHUNK, DPAD), jnp.float32),
            pltpu.SemaphoreType.DMA,
            pltpu.SemaphoreType.DMA,
            pltpu.SemaphoreType.DMA,
        ],
        compiler_params=pltpu.CompilerParams(
            use_tc_tiling_on_sc=False, needs_layout_passes=False
        ),
    )
    def gather_kernel(x_hbm, tab_hbm, out_hbm, tab_v, idx_v, rows_v, sem_t, sem_i, sem_o):
        c = lax.axis_index("c")
        s = lax.axis_index("s")
        wid = s * info.num_cores + c
        lanes = lax.iota(jnp.int32, 16)
        cols = [16 * j + lanes for j in range(NVR)]

        def base_of(k):
            return (wid + k * nw) * CHUNK

        def valid(k):
            return wid + k * nw < n_chunks

        def idx_copy(k):
            return pltpu.make_async_copy(
                x_hbm.at[pl.ds(base_of(k), CHUNK)], idx_v.at[k % 2], sem_i
            )

        def out_copy(k):
            return pltpu.make_async_copy(
                rows_v.at[k % 2], out_hbm.at[pl.ds(base_of(k), CHUNK)], sem_o
            )

        def when_start(k, cp):
            @pl.when(valid(k))
            def _():
                cp.start()

        def when_wait(k, cps):
            @pl.when(valid(k))
            def _():
                for cp in cps:
                    cp.wait()

        tab_cp = pltpu.make_async_copy(tab_hbm, tab_v, sem_t)
        tab_cp.start()
        when_start(0, idx_copy(0))
        tab_cp.wait()
        when_wait(0, [idx_copy(0)])
        if ni > 1:
            when_start(1, idx_copy(1))

        for i in range(ni):
            b = i % 2

            @pl.when(valid(i))
            def _(b=b):
                idx_b = idx_v.at[b]
                rows_b = rows_v.at[b]

                def body(g, carry):
                    a0 = g * UNROLL
                    xas = [
                        plsc.load_gather(idx_b, [lanes * 0 + (a0 + u)])
                        for u in range(UNROLL)
                    ]
                    for u in range(UNROLL):
                        a = a0 + u
                        for j in range(NVR):
                            v = plsc.load_gather(tab_v, [xas[u], cols[j]])
                            rows_b.at[a][pl.ds(16 * j, 16)] = v
                    return carry

                lax.fori_loop(0, CHUNK // UNROLL, body, 0)

            if i >= 1:
                when_wait(i - 1, [out_copy(i - 1)])
            when_start(i, out_copy(i))
            if i + 1 < ni:
                when_wait(i + 1, [idx_copy(i + 1)])
            if i + 2 < ni:
                when_start(i + 2, idx_copy(i + 2))

        when_wait(ni - 1, [out_copy(ni - 1)])

    return gather_kernel(x, tab)[:, :D]


# speedup vs baseline: 1.1289x; 1.0045x over previous
import functools

import jax
import jax.numpy as jnp
from jax import lax
from jax.experimental import pallas as pl
from jax.experimental.pallas import tpu as pltpu
from jax.experimental.pallas import tpu_sc as plsc

CHUNK = 400
DPAD = 96
NVR = DPAD // 16
UNROLL = 16


def kernel(x, atom_fea):
    B = x.shape[0]
    V, D = atom_fea.shape
    tab = jnp.pad(atom_fea, ((0, 0), (0, DPAD - D))).reshape(-1)
    n_chunks = B // CHUNK
    assert n_chunks * CHUNK == B

    info = plsc.get_sparse_core_info()
    nw = info.num_cores * info.num_subcores
    ni = (n_chunks + nw - 1) // nw
    mesh = plsc.VectorSubcoreMesh(core_axis_name="c", subcore_axis_name="s")

    @functools.partial(
        pl.kernel,
        mesh=mesh,
        out_type=jax.ShapeDtypeStruct((B, DPAD), jnp.float32),
        scratch_types=[
            pltpu.VMEM((V * DPAD,), jnp.float32),
            pltpu.VMEM((2, CHUNK), jnp.int32),
            pltpu.VMEM((2, CHUNK, DPAD), jnp.float32),
            pltpu.SemaphoreType.DMA,
            pltpu.SemaphoreType.DMA,
            pltpu.SemaphoreType.DMA,
        ],
        compiler_params=pltpu.CompilerParams(
            use_tc_tiling_on_sc=False, needs_layout_passes=False
        ),
    )
    def gather_kernel(x_hbm, tab_hbm, out_hbm, tab_v, idx_v, rows_v, sem_t, sem_i, sem_o):
        c = lax.axis_index("c")
        s = lax.axis_index("s")
        wid = s * info.num_cores + c
        lanes = lax.iota(jnp.int32, 16)
        cols = [16 * j + lanes for j in range(NVR)]

        def base_of(k):
            return (wid + k * nw) * CHUNK

        def valid(k):
            return wid + k * nw < n_chunks

        def idx_copy(k):
            return pltpu.make_async_copy(
                x_hbm.at[pl.ds(base_of(k), CHUNK)], idx_v.at[k % 2], sem_i
            )

        def out_copy(k):
            return pltpu.make_async_copy(
                rows_v.at[k % 2], out_hbm.at[pl.ds(base_of(k), CHUNK)], sem_o
            )

        def when_start(k, cp):
            @pl.when(valid(k))
            def _():
                cp.start()

        def when_wait(k, cps):
            @pl.when(valid(k))
            def _():
                for cp in cps:
                    cp.wait()

        tab_cp = pltpu.make_async_copy(tab_hbm, tab_v, sem_t)
        tab_cp.start()
        when_start(0, idx_copy(0))
        tab_cp.wait()
        when_wait(0, [idx_copy(0)])
        if ni > 1:
            when_start(1, idx_copy(1))

        for i in range(ni):
            b = i % 2

            @pl.when(valid(i))
            def _(b=b):
                idx_b = idx_v.at[b]
                rows_b = rows_v.at[b]

                def body(g, carry):
                    a0 = g * UNROLL
                    bases = idx_b[pl.ds(a0, UNROLL)] * DPAD
                    for u in range(UNROLL):
                        xa = lax.gather(
                            bases,
                            jnp.full((16, 1), u, jnp.int32),
                            lax.GatherDimensionNumbers(
                                offset_dims=(),
                                collapsed_slice_dims=(0,),
                                start_index_map=(0,),
                            ),
                            (1,),
                            mode=lax.GatherScatterMode.PROMISE_IN_BOUNDS,
                        )
                        for j in range(NVR):
                            v = plsc.load_gather(tab_v, [xa + cols[j]])
                            rows_b.at[a0 + u][pl.ds(16 * j, 16)] = v
                    return carry

                lax.fori_loop(0, CHUNK // UNROLL, body, 0)

            if i >= 1:
                when_wait(i - 1, [out_copy(i - 1)])
            when_start(i, out_copy(i))
            if i + 1 < ni:
                when_wait(i + 1, [idx_copy(i + 1)])
            if i + 2 < ni:
                when_start(i + 2, idx_copy(i + 2))

        when_wait(ni - 1, [out_copy(ni - 1)])

    return gather_kernel(x, tab)[:, :D]


# speedup vs baseline: 1.5247x; 1.3506x over previous
import functools

import jax
import jax.numpy as jnp
from jax import lax
from jax.experimental import pallas as pl
from jax.experimental.pallas import tpu as pltpu
from jax.experimental.pallas import tpu_sc as plsc

CHUNK = 400
DPAD = 96
NVR = DPAD // 16
UNROLL = 16


def kernel(x, atom_fea):
    B = x.shape[0]
    V, D = atom_fea.shape
    tab = jnp.pad(atom_fea, ((0, 0), (0, DPAD - D))).reshape(-1)
    n_chunks = B // CHUNK
    assert n_chunks * CHUNK == B

    info = plsc.get_sparse_core_info()
    nw = info.num_cores * info.num_subcores
    ni = (n_chunks + nw - 1) // nw
    mesh = plsc.VectorSubcoreMesh(core_axis_name="c", subcore_axis_name="s")

    @functools.partial(
        pl.kernel,
        mesh=mesh,
        out_type=jax.ShapeDtypeStruct((B, DPAD), jnp.float32),
        scratch_types=[
            pltpu.VMEM((V * DPAD,), jnp.float32),
            pltpu.VMEM((2, CHUNK), jnp.int32),
            pltpu.VMEM((2, CHUNK, DPAD), jnp.float32),
            pltpu.SemaphoreType.DMA,
            pltpu.SemaphoreType.DMA,
            pltpu.SemaphoreType.DMA,
        ],
        compiler_params=pltpu.CompilerParams(
            use_tc_tiling_on_sc=False, needs_layout_passes=False
        ),
    )
    def gather_kernel(x_hbm, tab_hbm, out_hbm, tab_v, idx_v, rows_v, sem_t, sem_i, sem_o):
        c = lax.axis_index("c")
        s = lax.axis_index("s")
        wid = s * info.num_cores + c
        lanes = lax.iota(jnp.int32, 16)
        cols = [16 * j + lanes for j in range(NVR)]

        def base_of(k):
            return (wid + k * nw) * CHUNK

        def valid(k):
            return wid + k * nw < n_chunks

        def idx_copy(k):
            return pltpu.make_async_copy(
                x_hbm.at[pl.ds(base_of(k), CHUNK)], idx_v.at[k % 2], sem_i
            )

        def out_copy(k):
            return pltpu.make_async_copy(
                rows_v.at[k % 2], out_hbm.at[pl.ds(base_of(k), CHUNK)], sem_o
            )

        def when_start(k, cp):
            @pl.when(valid(k))
            def _():
                cp.start()

        def when_wait(k, cps):
            @pl.when(valid(k))
            def _():
                for cp in cps:
                    cp.wait()

        tab_cp = pltpu.make_async_copy(tab_hbm, tab_v, sem_t)
        tab_cp.start()
        when_start(0, idx_copy(0))
        tab_cp.wait()
        when_wait(0, [idx_copy(0)])
        if ni > 1:
            when_start(1, idx_copy(1))

        for i in range(ni):
            b = i % 2

            @pl.when(valid(i))
            def _(b=b):
                idx_b = idx_v.at[b]
                rows_b = rows_v.at[b]

                def lane_broadcast(vec, u):
                    return lax.gather(
                        vec,
                        jnp.full((16, 1), u, jnp.int32),
                        lax.GatherDimensionNumbers(
                            offset_dims=(),
                            collapsed_slice_dims=(0,),
                            start_index_map=(0,),
                        ),
                        (1,),
                        mode=lax.GatherScatterMode.PROMISE_IN_BOUNDS,
                    )

                def body(g, carry):
                    a0 = g * UNROLL
                    bases = idx_b[pl.ds(a0, UNROLL)] * DPAD
                    prev = None
                    for u in range(UNROLL + 1):
                        cur = None
                        if u < UNROLL:
                            xa = lane_broadcast(bases, u)
                            cur = [
                                plsc.load_gather(tab_v, [xa + cols[j]])
                                for j in range(NVR)
                            ]
                        if prev is not None:
                            for j in range(NVR):
                                rows_b.at[a0 + u - 1][pl.ds(16 * j, 16)] = prev[j]
                        prev = cur
                    return carry

                lax.fori_loop(0, CHUNK // UNROLL, body, 0)

            if i >= 1:
                when_wait(i - 1, [out_copy(i - 1)])
            when_start(i, out_copy(i))
            if i + 1 < ni:
                when_wait(i + 1, [idx_copy(i + 1)])
            if i + 2 < ni:
                when_start(i + 2, idx_copy(i + 2))

        when_wait(ni - 1, [out_copy(ni - 1)])

    return gather_kernel(x, tab)[:, :D]


# speedup vs baseline: 1.5463x; 1.0142x over previous
import functools

import jax
import jax.numpy as jnp
from jax import lax
from jax.experimental import pallas as pl
from jax.experimental.pallas import tpu as pltpu
from jax.experimental.pallas import tpu_sc as plsc

CHUNK = 400
DPAD = 96
NVR = DPAD // 16
UNROLL = 16


def kernel(x, atom_fea):
    B = x.shape[0]
    V, D = atom_fea.shape
    tab = jnp.pad(atom_fea, ((0, 0), (0, DPAD - D))).reshape(-1)
    n_chunks = B // CHUNK
    assert n_chunks * CHUNK == B

    info = plsc.get_sparse_core_info()
    nw = info.num_cores * info.num_subcores
    ni = (n_chunks + nw - 1) // nw
    mesh = plsc.VectorSubcoreMesh(core_axis_name="c", subcore_axis_name="s")

    @functools.partial(
        pl.kernel,
        mesh=mesh,
        out_type=jax.ShapeDtypeStruct((B, DPAD), jnp.float32),
        scratch_types=[
            pltpu.VMEM((V * DPAD,), jnp.float32),
            pltpu.VMEM((2, CHUNK), jnp.int32),
            pltpu.VMEM((2, CHUNK, DPAD), jnp.float32),
            pltpu.SemaphoreType.DMA,
            pltpu.SemaphoreType.DMA,
            pltpu.SemaphoreType.DMA,
        ],
        compiler_params=pltpu.CompilerParams(
            use_tc_tiling_on_sc=False, needs_layout_passes=False
        ),
    )
    def gather_kernel(x_hbm, tab_hbm, out_hbm, tab_v, idx_v, rows_v, sem_t, sem_i, sem_o):
        c = lax.axis_index("c")
        s = lax.axis_index("s")
        wid = s * info.num_cores + c
        lanes = lax.iota(jnp.int32, 16)
        tab_views = [
            tab_v.at[pl.ds(16 * j, V * DPAD - 16 * j)] for j in range(NVR)
        ]

        def base_of(k):
            return (wid + k * nw) * CHUNK

        def valid(k):
            return wid + k * nw < n_chunks

        def idx_copy(k):
            return pltpu.make_async_copy(
                x_hbm.at[pl.ds(base_of(k), CHUNK)], idx_v.at[k % 2], sem_i
            )

        def out_copy(k):
            return pltpu.make_async_copy(
                rows_v.at[k % 2], out_hbm.at[pl.ds(base_of(k), CHUNK)], sem_o
            )

        def when_start(k, cp):
            @pl.when(valid(k))
            def _():
                cp.start()

        def when_wait(k, cps):
            @pl.when(valid(k))
            def _():
                for cp in cps:
                    cp.wait()

        tab_cp = pltpu.make_async_copy(tab_hbm, tab_v, sem_t)
        tab_cp.start()
        when_start(0, idx_copy(0))
        tab_cp.wait()
        when_wait(0, [idx_copy(0)])
        if ni > 1:
            when_start(1, idx_copy(1))

        for i in range(ni):
            b = i % 2

            @pl.when(valid(i))
            def _(b=b):
                idx_b = idx_v.at[b]
                rows_b = rows_v.at[b]

                def lane_broadcast(vec, u):
                    return lax.gather(
                        vec,
                        jnp.full((16, 1), u, jnp.int32),
                        lax.GatherDimensionNumbers(
                            offset_dims=(),
                            collapsed_slice_dims=(0,),
                            start_index_map=(0,),
                        ),
                        (1,),
                        mode=lax.GatherScatterMode.PROMISE_IN_BOUNDS,
                    )

                @plsc.parallel_loop(0, CHUNK // UNROLL)
                def body(g):
                    a0 = g * UNROLL
                    bases = idx_b[pl.ds(a0, UNROLL)] * DPAD
                    prev = None
                    for u in range(UNROLL + 1):
                        cur = None
                        if u < UNROLL:
                            xi16 = lane_broadcast(bases, u) + lanes
                            cur = [
                                plsc.load_gather(tab_views[j], [xi16])
                                for j in range(NVR)
                            ]
                        if prev is not None:
                            for j in range(NVR):
                                rows_b.at[a0 + u - 1][pl.ds(16 * j, 16)] = prev[j]
                        prev = cur

            if i >= 1:
                when_wait(i - 1, [out_copy(i - 1)])
            when_start(i, out_copy(i))
            if i + 1 < ni:
                when_wait(i + 1, [idx_copy(i + 1)])
            if i + 2 < ni:
                when_start(i + 2, idx_copy(i + 2))

        when_wait(ni - 1, [out_copy(ni - 1)])

    return gather_kernel(x, tab)[:, :D]
